# Initial kernel scaffold; baseline (speedup 1.0000x reference)
import jax
import jax.numpy as jnp
from jax.experimental import pallas as pl


def kernel(user_feats, item_feats, edge_index_buys, edge_index_bought, W_buys, b_buys, W_bought, b_bought):
    raise NotImplementedError("write your pallas kernel here")


# speedup vs baseline: 4.2370x; 4.2370x over previous
import functools

import jax
import jax.numpy as jnp
from jax import lax
from jax.experimental import pallas as pl
from jax.experimental.pallas import tpu as pltpu
from jax.experimental.pallas import tpu_sc as plsc

N_NODES = 50000
D_IN = 128
D_OUT = 64
E_EDGES = 800000

NC = 2
NS = 16
L = 16

HALF = N_NODES // NC
ROWS_PER_TILE = 1568
PAD = ROWS_PER_TILE * NS
TRASH = HALF

K = 128
EPT = 50048
E_PADDED = EPT * NS
N_CHUNKS = EPT // K

FIN = 200
FIN_CHUNKS = HALF // FIN


def _matmul_bias(x, w, b):
  m, kdim = x.shape
  n = w.shape[1]
  bm = 1000

  def body(x_ref, w_ref, b_ref, o_ref):
    o_ref[...] = (
        jnp.dot(x_ref[...], w_ref[...], preferred_element_type=jnp.float32)
        + b_ref[...]
    )

  return pl.pallas_call(
      body,
      grid=(m // bm,),
      in_specs=[
          pl.BlockSpec((bm, kdim), lambda i: (i, 0)),
          pl.BlockSpec((kdim, n), lambda i: (0, 0)),
          pl.BlockSpec((1, n), lambda i: (0, 0)),
      ],
      out_specs=pl.BlockSpec((bm, n), lambda i: (i, 0)),
      out_shape=jax.ShapeDtypeStruct((m, n), jnp.float32),
  )(x, w, b.reshape(1, n))


def _agg_body(wh_hbm, src_hbm, dst_hbm, out_hbm,
              acc_sh, deg_sh, sidx_v, ldst_v, rows_v, ones_v, zline_v,
              fin_v, fdeg_v, sem):
  c = lax.axis_index("c")
  s = lax.axis_index("s")
  lo = (c * HALF).astype(jnp.int32)

  zeros16 = jnp.zeros((L,), jnp.float32)
  ones16 = jnp.ones((L,), jnp.float32)

  def zrow(r, carry):
    for j in range(D_OUT // L):
      rows_v[r, pl.ds(j * L, L)] = zeros16
    return carry
  lax.fori_loop(0, K, zrow, 0)

  def zline(r, carry):
    zline_v[pl.ds(r * L, L)] = zeros16
    return carry
  lax.fori_loop(0, ROWS_PER_TILE // L, zline, 0)

  for j in range(K // L):
    ones_v[pl.ds(j * L, L)] = ones16

  base_r = s * ROWS_PER_TILE
  for j in range(12):
    pltpu.sync_copy(rows_v, acc_sh.at[pl.ds(base_r + j * K, K)])
  pltpu.sync_copy(rows_v.at[pl.ds(0, 32)],
                  acc_sh.at[pl.ds(base_r + 12 * K, 32)])
  pltpu.sync_copy(zline_v, deg_sh.at[pl.ds(base_r, ROWS_PER_TILE)])

  plsc.subcore_barrier()

  def edge_chunk(g, carry):
    base_e = pl.multiple_of(s * EPT + g * K, 8)
    pltpu.sync_copy(src_hbm.at[pl.ds(base_e, K)], sidx_v)
    pltpu.sync_copy(dst_hbm.at[pl.ds(base_e, K)], ldst_v)
    pltpu.async_copy(wh_hbm.at[sidx_v], rows_v, sem).wait()
    for j in range(K // L):
      d = ldst_v[pl.ds(j * L, L)]
      ok = (d >= lo) & (d < lo + HALF)
      ldst_v[pl.ds(j * L, L)] = jnp.where(ok, d - lo, TRASH)
    pltpu.sync_copy(rows_v, acc_sh.at[ldst_v], add=True)
    pltpu.sync_copy(ones_v, deg_sh.at[ldst_v], add=True)
    return carry
  lax.fori_loop(0, N_CHUNKS, edge_chunk, 0)

  plsc.subcore_barrier()

  def fin_chunk(cid):
    r0 = cid * FIN
    pltpu.sync_copy(acc_sh.at[pl.ds(r0, FIN)], fin_v)
    pltpu.sync_copy(deg_sh.at[pl.ds(r0, FIN)], fdeg_v.at[pl.ds(0, FIN)])

    def div_row(r, carry):
      dv = fdeg_v[pl.ds(r, L)]
      dvv = jnp.full((L,), dv[0], jnp.float32)
      invv = 1.0 / jnp.maximum(dvv, 1.0)
      for j in range(D_OUT // L):
        fin_v[r, pl.ds(j * L, L)] = fin_v[r, pl.ds(j * L, L)] * invv
      return carry
    lax.fori_loop(0, FIN, div_row, 0)
    pltpu.sync_copy(fin_v, out_hbm.at[pl.ds(lo + r0, FIN)])

  def fin_loop(kk, carry):
    cid = s + kk * NS
    fin_chunk(cid)
    return carry
  lax.fori_loop(0, 7, fin_loop, 0)

  @pl.when(s < FIN_CHUNKS - 112)
  def _():
    fin_chunk(112 + s)


def _aggregate(wh, src, dst):
  mesh = plsc.VectorSubcoreMesh(
      core_axis_name="c", subcore_axis_name="s",
      num_cores=NC, num_subcores=NS)

  k = functools.partial(
      pl.kernel,
      out_type=jax.ShapeDtypeStruct((N_NODES, D_OUT), jnp.float32),
      mesh=mesh,
      compiler_params=pltpu.CompilerParams(use_tc_tiling_on_sc=False),
      scratch_types=[
          pltpu.VMEM_SHARED((PAD, D_OUT), jnp.float32),
          pltpu.VMEM_SHARED((PAD,), jnp.float32),
          pltpu.VMEM((K,), jnp.int32),
          pltpu.VMEM((K,), jnp.int32),
          pltpu.VMEM((K, D_OUT), jnp.float32),
          pltpu.VMEM((K,), jnp.float32),
          pltpu.VMEM((ROWS_PER_TILE,), jnp.float32),
          pltpu.VMEM((FIN, D_OUT), jnp.float32),
          pltpu.VMEM((FIN + L,), jnp.float32),
          pltpu.SemaphoreType.DMA,
      ],
  )(_agg_body)
  return k(wh, src, dst)


def kernel(user_feats, item_feats, edge_index_buys, edge_index_bought,
           W_buys, b_buys, W_bought, b_bought):
  wh_buys = _matmul_bias(user_feats, W_buys, b_buys)
  wh_bought = _matmul_bias(item_feats, W_bought, b_bought)

  pad_n = E_PADDED - E_EDGES
  pad_src = jnp.zeros((pad_n,), jnp.int32)
  pad_dst = jnp.full((pad_n,), N_NODES, jnp.int32)

  src_buys = jnp.concatenate(
      [edge_index_buys[0].astype(jnp.int32), pad_src])
  dst_buys = jnp.concatenate(
      [edge_index_buys[1].astype(jnp.int32), pad_dst])
  src_bought = jnp.concatenate(
      [edge_index_bought[0].astype(jnp.int32), pad_src])
  dst_bought = jnp.concatenate(
      [edge_index_bought[1].astype(jnp.int32), pad_dst])

  h_item = _aggregate(wh_buys, src_buys, dst_buys)
  h_user = _aggregate(wh_bought, src_bought, dst_bought)
  return (h_user, h_item)


# speedup vs baseline: 5.6189x; 1.3261x over previous
import functools

import jax
import jax.numpy as jnp
from jax import lax
from jax.experimental import pallas as pl
from jax.experimental.pallas import tpu as pltpu
from jax.experimental.pallas import tpu_sc as plsc

N_NODES = 50000
D_IN = 128
D_OUT = 64
E_EDGES = 800000

NC = 2
NS = 16
L = 16

HALF = N_NODES // NC
ROWS_PER_TILE = 1568
PAD = ROWS_PER_TILE * NS
TRASH = HALF

K = 128
CPB = 14
BLOCK = CPB * K
NBLK = 28
EPT = BLOCK * NBLK
E_PADDED = EPT * NS
ROWS_2D = E_PADDED // K

FIN = 40
FIN_CHUNKS = HALF // FIN


def _matmul_bias(x, w, b):
  m, kdim = x.shape
  n = w.shape[1]
  bm = 1000

  def body(x_ref, w_ref, b_ref, o_ref):
    o_ref[...] = (
        jnp.dot(x_ref[...], w_ref[...], preferred_element_type=jnp.float32)
        + b_ref[...]
    )

  return pl.pallas_call(
      body,
      grid=(m // bm,),
      in_specs=[
          pl.BlockSpec((bm, kdim), lambda i: (i, 0)),
          pl.BlockSpec((kdim, n), lambda i: (0, 0)),
          pl.BlockSpec((1, n), lambda i: (0, 0)),
      ],
      out_specs=pl.BlockSpec((bm, n), lambda i: (i, 0)),
      out_shape=jax.ShapeDtypeStruct((m, n), jnp.float32),
  )(x, w, b.reshape(1, n))


def _agg_body(wh_hbm, src_hbm, dst_hbm, out_hbm,
              acc_sh, deg_sh, sblk_v, ldst_v, rows0_v, rows1_v, ones_v,
              zline_v,
              sem_g0, sem_g1, sem_s0, sem_s1, sem_d0, sem_d1):
  c = lax.axis_index("c")
  s = lax.axis_index("s")
  lo = (c * HALF).astype(jnp.int32)

  zeros16 = jnp.zeros((L,), jnp.float32)
  ones16 = jnp.ones((L,), jnp.float32)
  rows_v = (rows0_v, rows1_v)
  sem_g = (sem_g0, sem_g1)
  sem_s = (sem_s0, sem_s1)
  sem_d = (sem_d0, sem_d1)

  def zrow(r, carry):
    for j in range(D_OUT // L):
      rows0_v[r, pl.ds(j * L, L)] = zeros16
    return carry
  lax.fori_loop(0, K, zrow, 0)

  def zline(r, carry):
    zline_v[pl.ds(r * L, L)] = zeros16
    return carry
  lax.fori_loop(0, ROWS_PER_TILE // L, zline, 0)

  for j in range(K // L):
    ones_v[pl.ds(j * L, L)] = ones16

  base_r = s * ROWS_PER_TILE
  for j in range(12):
    pltpu.sync_copy(rows0_v, acc_sh.at[pl.ds(base_r + j * K, K)])
  pltpu.sync_copy(rows0_v.at[pl.ds(0, 32)],
                  acc_sh.at[pl.ds(base_r + 12 * K, 32)])
  pltpu.sync_copy(zline_v, deg_sh.at[pl.ds(base_r, ROWS_PER_TILE)])

  plsc.subcore_barrier()

  def block_body(blk, carry):
    rb = s * (EPT // K) + blk * CPB
    pltpu.sync_copy(src_hbm.at[pl.ds(rb, CPB)], sblk_v)
    pltpu.sync_copy(dst_hbm.at[pl.ds(rb, CPB)], ldst_v)

    g0 = pltpu.async_copy(wh_hbm.at[sblk_v.at[0]], rows0_v, sem_g[0])
    g1 = pltpu.async_copy(wh_hbm.at[sblk_v.at[1]], rows1_v, sem_g[1])
    gathers = [g0, g1]

    for g in range(CPB):
      for j in range(K // L):
        d = ldst_v[g, pl.ds(j * L, L)]
        ok = (d >= lo) & (d < lo + HALF)
        ldst_v[g, pl.ds(j * L, L)] = jnp.where(ok, d - lo, TRASH)

    for g in range(CPB):
      b = g % 2
      gathers[b].wait()
      sd = pltpu.async_copy(rows_v[b], acc_sh.at[ldst_v.at[g]],
                            sem_s[b], add=True)
      dd = pltpu.async_copy(ones_v, deg_sh.at[ldst_v.at[g]],
                            sem_d[b], add=True)
      sd.wait()
      dd.wait()
      if g + 2 < CPB:
        gathers[b] = pltpu.async_copy(
            wh_hbm.at[sblk_v.at[g + 2]], rows_v[b], sem_g[b])
    return carry
  lax.fori_loop(0, NBLK, block_body, 0)

  plsc.subcore_barrier()

  def fin_chunk(cid):
    r0 = cid * FIN
    pltpu.sync_copy(acc_sh.at[pl.ds(r0, FIN)], rows0_v.at[pl.ds(0, FIN)])
    pltpu.sync_copy(deg_sh.at[pl.ds(r0, FIN)], zline_v.at[pl.ds(0, FIN)])

    def div_row(r, carry):
      dv = zline_v[pl.ds(r, L)]
      dvv = jnp.full((L,), dv[0], jnp.float32)
      invv = 1.0 / jnp.maximum(dvv, 1.0)
      for j in range(D_OUT // L):
        rows0_v[r, pl.ds(j * L, L)] = rows0_v[r, pl.ds(j * L, L)] * invv
      return carry
    lax.fori_loop(0, FIN, div_row, 0)
    pltpu.sync_copy(rows0_v.at[pl.ds(0, FIN)], out_hbm.at[pl.ds(lo + r0, FIN)])

  def fin_loop(kk, carry):
    cid = s + kk * NS
    fin_chunk(cid)
    return carry
  lax.fori_loop(0, 39, fin_loop, 0)

  @pl.when(s < FIN_CHUNKS - 624)
  def _():
    fin_chunk(624 + s)


def _aggregate(wh, src2d, dst2d):
  mesh = plsc.VectorSubcoreMesh(
      core_axis_name="c", subcore_axis_name="s",
      num_cores=NC, num_subcores=NS)

  k = functools.partial(
      pl.kernel,
      out_type=jax.ShapeDtypeStruct((N_NODES, D_OUT), jnp.float32),
      mesh=mesh,
      compiler_params=pltpu.CompilerParams(use_tc_tiling_on_sc=False),
      scratch_types=[
          pltpu.VMEM_SHARED((PAD, D_OUT), jnp.float32),
          pltpu.VMEM_SHARED((PAD,), jnp.float32),
          pltpu.VMEM((CPB, K), jnp.int32),
          pltpu.VMEM((CPB, K), jnp.int32),
          pltpu.VMEM((K, D_OUT), jnp.float32),
          pltpu.VMEM((K, D_OUT), jnp.float32),
          pltpu.VMEM((K,), jnp.float32),
          pltpu.VMEM((ROWS_PER_TILE,), jnp.float32),
          pltpu.SemaphoreType.DMA,
          pltpu.SemaphoreType.DMA,
          pltpu.SemaphoreType.DMA,
          pltpu.SemaphoreType.DMA,
          pltpu.SemaphoreType.DMA,
          pltpu.SemaphoreType.DMA,
      ],
  )(_agg_body)
  return k(wh, src2d, dst2d)


def kernel(user_feats, item_feats, edge_index_buys, edge_index_bought,
           W_buys, b_buys, W_bought, b_bought):
  wh_buys = _matmul_bias(user_feats, W_buys, b_buys)
  wh_bought = _matmul_bias(item_feats, W_bought, b_bought)

  pad_n = E_PADDED - E_EDGES
  pad_src = jnp.zeros((pad_n,), jnp.int32)
  pad_dst = jnp.full((pad_n,), N_NODES, jnp.int32)

  def prep(ei):
    src = jnp.concatenate([ei[0].astype(jnp.int32), pad_src])
    dst = jnp.concatenate([ei[1].astype(jnp.int32), pad_dst])
    return src.reshape(ROWS_2D, K), dst.reshape(ROWS_2D, K)

  src_buys, dst_buys = prep(edge_index_buys)
  src_bought, dst_bought = prep(edge_index_bought)

  h_item = _aggregate(wh_buys, src_buys, dst_buys)
  h_user = _aggregate(wh_bought, src_bought, dst_bought)
  return (h_user, h_item)
